# Initial kernel scaffold; baseline (speedup 1.0000x reference)
import jax
import jax.numpy as jnp
from jax.experimental import pallas as pl


def kernel(z, W, emb):
    raise NotImplementedError("write your pallas kernel here")


# speedup vs baseline: 1.7122x; 1.7122x over previous
import jax
import jax.numpy as jnp
from jax import lax
from jax.experimental import pallas as pl


B, C_IN, N = 4, 384, 576
D, K = 64, 512


def _vq_tc_body(z_ref, w_ref, emb_ref, md_ref, mi_ref, zn_ref, en_ref):
    emb = emb_ref[...]
    emb_sq = jnp.sum(emb * emb, axis=1, keepdims=True)
    en_ref[...] = jnp.sqrt(emb_sq)
    w = w_ref[...].astype(jnp.bfloat16)
    iota_k = lax.broadcasted_iota(jnp.int32, (K, N), 0)
    for b in range(B):
        zb = z_ref[b].astype(jnp.bfloat16)
        ze = jnp.dot(w, zb, preferred_element_type=jnp.float32)
        g = jnp.dot(emb, ze, preferred_element_type=jnp.float32,
                    precision=lax.Precision.HIGHEST)
        s = emb_sq - 2.0 * g
        s_min = jnp.min(s, axis=0, keepdims=True)
        ind = jnp.min(jnp.where(s == s_min, iota_k, K), axis=0,
                      keepdims=True)
        ze_sq = jnp.sum(ze * ze, axis=0, keepdims=True)
        md_ref[b, :] = (s_min + ze_sq)[0]
        mi_ref[b, :] = ind[0]
        zn_ref[b, :] = jnp.sqrt(ze_sq)[0]


def _vq_tc(z, W, emb):
    return pl.pallas_call(
        _vq_tc_body,
        out_shape=[
            jax.ShapeDtypeStruct((B, N), jnp.float32),
            jax.ShapeDtypeStruct((B, N), jnp.int32),
            jax.ShapeDtypeStruct((B, N), jnp.float32),
            jax.ShapeDtypeStruct((K, 1), jnp.float32),
        ],
    )(z, W, emb)


def kernel(z, W, emb):
    min_dist, min_ind, ze_norm, emb_norm = _vq_tc(z, W, emb)
    zq = jnp.transpose(emb[min_ind], (0, 2, 1))
    ind_hist = jnp.bincount(min_ind.reshape(-1), length=K).astype(jnp.float32)
    return zq, min_dist, ind_hist, ze_norm, emb_norm.reshape(K)
